# baseline (speedup 1.0000x reference)
import functools

import jax
import jax.numpy as jnp
from jax.experimental import pallas as pl
from jax.experimental.pallas import tpu as pltpu

_BATCH = 8192
_D = 2048
_N_CLUSTERS = 3
_ITERS = 20
_EPS = 1e-7
_BAND = 0.01
_BR = 512
_NB = _BATCH // _BR


def _body(state0_ref, u_ref, w_ref, b_ref, z_ref,
          fz_ref, ld_ref, pen_ref,
          state_ref, total_ref, count_ref, colsum_ref):
    t = pl.program_id(0)
    i = pl.program_id(1)

    @pl.when((t == 0) & (i == 0))
    def _init():
        state_ref[...] = state0_ref[...]
        total_ref[...] = jnp.zeros_like(total_ref)
        count_ref[...] = jnp.zeros_like(count_ref)
        colsum_ref[...] = jnp.zeros_like(colsum_ref)

    zb = z_ref[...]
    st = state_ref[...]

    @pl.when(t == 0)
    def _colsum():
        colsum_ref[...] += jnp.sum(zb, axis=0, keepdims=True)

    @pl.when(t < _ITERS)
    def _kmeans_step():
        scores = jax.lax.dot_general(
            zb.astype(jnp.bfloat16), st.astype(jnp.bfloat16),
            (((1,), (1,)), ((), ())),
            preferred_element_type=jnp.float32)
        d0 = jnp.sum(st[0:1, :] * st[0:1, :]) - 2.0 * scores[:, 0:1]
        d1 = jnp.sum(st[1:2, :] * st[1:2, :]) - 2.0 * scores[:, 1:2]
        d2 = jnp.sum(st[2:3, :] * st[2:3, :]) - 2.0 * scores[:, 2:3]
        m0 = (d0 <= d1) & (d0 <= d2)
        m1 = (d1 < d0) & (d1 <= d2)
        acc0 = jnp.zeros((8, _D), jnp.float32)
        acc1 = jnp.zeros((8, _D), jnp.float32)
        for g in range(0, _BR, 8):
            sub = zb[g:g + 8, :]
            acc0 = acc0 + jnp.where(m0[g:g + 8, :], sub, 0.0)
            acc1 = acc1 + jnp.where(m1[g:g + 8, :], sub, 0.0)
        total_ref[0:1, :] += jnp.sum(acc0, axis=0, keepdims=True)
        total_ref[1:2, :] += jnp.sum(acc1, axis=0, keepdims=True)
        count_ref[0:1, :] += jnp.sum(m0.astype(jnp.float32))
        count_ref[1:2, :] += jnp.sum(m1.astype(jnp.float32))

        @pl.when(i == _NB - 1)
        def _update_centers():
            t0 = total_ref[0:1, :]
            t1 = total_ref[1:2, :]
            t2 = colsum_ref[...] - t0 - t1
            c0 = count_ref[0:1, 0:1]
            c1 = count_ref[1:2, 0:1]
            c2 = float(_BATCH) - c0 - c1
            state_ref[0:1, :] = t0 / c0
            state_ref[1:2, :] = t1 / c1
            state_ref[2:3, :] = t2 / c2
            total_ref[...] = jnp.zeros_like(total_ref)
            count_ref[...] = jnp.zeros_like(count_ref)

    @pl.when(t == _ITERS)
    def _final():
        z_sq = jnp.sum(zb * zb, axis=1, keepdims=True)
        sc = jax.lax.dot_general(
            zb, st, (((1,), (1,)), ((), ())),
            preferred_element_type=jnp.float32)
        n0 = jnp.sqrt(jnp.maximum(
            z_sq - 2.0 * sc[:, 0:1] + jnp.sum(st[0:1, :] * st[0:1, :]), 0.0))
        n1 = jnp.sqrt(jnp.maximum(
            z_sq - 2.0 * sc[:, 1:2] + jnp.sum(st[1:2, :] * st[1:2, :]), 0.0))
        n2 = jnp.sqrt(jnp.maximum(
            z_sq - 2.0 * sc[:, 2:3] + jnp.sum(st[2:3, :] * st[2:3, :]), 0.0))
        measure = jnp.minimum(jnp.minimum(n0, n1), n2)
        m2 = measure * measure
        c_base = 2.0 * _D
        beta = jnp.zeros_like(m2)
        for scale in (0.1, 0.2, 0.5, 1.0, 2.0, 5.0, 10.0):
            c = c_base * scale
            beta = beta + c / (c + m2)
        pen_ref[...] = _BAND * jnp.log(jnp.abs(beta) + _EPS)

        u = u_ref[...]
        w = w_ref[...]
        uw = jnp.sum(u * w)
        muw = -1.0 + jax.nn.softplus(uw)
        uhat = u + (muw - uw) * w / jnp.sum(w * w)
        zwb = jnp.sum(zb * w, axis=1, keepdims=True) + b_ref[0, 0]
        th = jnp.tanh(zwb)
        fz_ref[...] = zb + th * uhat
        wu = jnp.sum(w * uhat)
        psi_u = (1.0 - th * th) * wu
        ld_ref[...] = jnp.log(jnp.abs(1.0 + psi_u) + _EPS)


@functools.partial(jax.jit, static_argnames=("interpret",))
def _run(z, u, w, b, interpret=False):
    p = jax.random.uniform(jax.random.key(42), (z.shape[0],),
                           minval=0.0, maxval=1.0)
    _, ind = jax.lax.top_k(p, _N_CLUSTERS)
    state0 = jnp.take(z, ind, axis=0)

    u2 = u.reshape(1, _D)
    w2 = w.reshape(1, _D)
    b2 = b.reshape(1, 1)

    def _const_map(t, i):
        return (0, 0)

    def _z_map(t, i):
        return (i, 0)

    def _out_map(t, i):
        return (jnp.where(t == _ITERS, i, 0), 0)

    fz, ld, pen = pl.pallas_call(
        _body,
        grid=(_ITERS + 1, _NB),
        in_specs=[
            pl.BlockSpec((_N_CLUSTERS, _D), _const_map),
            pl.BlockSpec((1, _D), _const_map),
            pl.BlockSpec((1, _D), _const_map),
            pl.BlockSpec((1, 1), _const_map),
            pl.BlockSpec((_BR, _D), _z_map),
        ],
        out_specs=[
            pl.BlockSpec((_BR, _D), _out_map),
            pl.BlockSpec((_BR, 1), _out_map),
            pl.BlockSpec((_BR, 1), _out_map),
        ],
        out_shape=[
            jax.ShapeDtypeStruct((_BATCH, _D), jnp.float32),
            jax.ShapeDtypeStruct((_BATCH, 1), jnp.float32),
            jax.ShapeDtypeStruct((_BATCH, 1), jnp.float32),
        ],
        scratch_shapes=[
            pltpu.VMEM((_N_CLUSTERS, _D), jnp.float32),
            pltpu.VMEM((_N_CLUSTERS, _D), jnp.float32),
            pltpu.VMEM((_N_CLUSTERS, 128), jnp.float32),
            pltpu.VMEM((1, _D), jnp.float32),
        ],
        compiler_params=pltpu.CompilerParams(
            dimension_semantics=("arbitrary", "arbitrary"),
        ),
        interpret=interpret,
    )(state0, u2, w2, b2, z)
    return fz, ld.reshape(-1), pen.reshape(-1)


def kernel(z, u, w, b):
    return _run(z, u, w, b)


# speedup vs baseline: 1.0804x; 1.0804x over previous
import functools

import jax
import jax.numpy as jnp
from jax.experimental import pallas as pl
from jax.experimental.pallas import tpu as pltpu

_BATCH = 8192
_D = 2048
_N_CLUSTERS = 3
_ITERS = 20
_EPS = 1e-7
_BAND = 0.01
_BR = 512
_NB = _BATCH // _BR


def _kmeans_step(zf, zbf, st, i, state_ref, total_ref, count_ref, colsum_ref):
    scores = jax.lax.dot_general(
        zbf, st.astype(jnp.bfloat16), (((1,), (1,)), ((), ())),
        preferred_element_type=jnp.float32)
    d0 = jnp.sum(st[0:1, :] * st[0:1, :]) - 2.0 * scores[:, 0:1]
    d1 = jnp.sum(st[1:2, :] * st[1:2, :]) - 2.0 * scores[:, 1:2]
    d2 = jnp.sum(st[2:3, :] * st[2:3, :]) - 2.0 * scores[:, 2:3]
    m0 = (d0 <= d1) & (d0 <= d2)
    m1 = (d1 < d0) & (d1 <= d2)
    acc0 = jnp.zeros((8, _D), jnp.float32)
    acc1 = jnp.zeros((8, _D), jnp.float32)
    for g in range(0, _BR, 8):
        sub = zf[g:g + 8, :]
        acc0 = acc0 + jnp.where(m0[g:g + 8, :], sub, 0.0)
        acc1 = acc1 + jnp.where(m1[g:g + 8, :], sub, 0.0)
    total_ref[0:1, :] += jnp.sum(acc0, axis=0, keepdims=True)
    total_ref[1:2, :] += jnp.sum(acc1, axis=0, keepdims=True)
    count_ref[0:1, :] += jnp.sum(m0.astype(jnp.float32))
    count_ref[1:2, :] += jnp.sum(m1.astype(jnp.float32))

    @pl.when(i == _NB - 1)
    def _update_centers():
        t0 = total_ref[0:1, :]
        t1 = total_ref[1:2, :]
        t2 = colsum_ref[...] - t0 - t1
        c0 = count_ref[0:1, 0:1]
        c1 = count_ref[1:2, 0:1]
        c2 = float(_BATCH) - c0 - c1
        state_ref[0:1, :] = t0 / c0
        state_ref[1:2, :] = t1 / c1
        state_ref[2:3, :] = t2 / c2
        total_ref[...] = jnp.zeros_like(total_ref)
        count_ref[...] = jnp.zeros_like(count_ref)


def _body(state0_ref, u_ref, w_ref, b_ref, z_ref,
          fz_ref, ld_ref, pen_ref,
          state_ref, total_ref, count_ref, colsum_ref, res_ref):
    t = pl.program_id(0)
    i = pl.program_id(1)

    @pl.when((t == 0) & (i == 0))
    def _init():
        state_ref[...] = state0_ref[...]
        total_ref[...] = jnp.zeros_like(total_ref)
        count_ref[...] = jnp.zeros_like(count_ref)
        colsum_ref[...] = jnp.zeros_like(colsum_ref)

    st = state_ref[...]

    @pl.when(t == 0)
    def _first_iter():
        zb = z_ref[...]
        zbf = zb.astype(jnp.bfloat16)
        res_ref[i] = zbf
        colsum_ref[...] += jnp.sum(zb, axis=0, keepdims=True)
        _kmeans_step(zb, zbf, st, i, state_ref, total_ref, count_ref,
                     colsum_ref)

    @pl.when((t > 0) & (t < _ITERS))
    def _resident_iter():
        zbf = res_ref[i]
        zf = zbf.astype(jnp.float32)
        _kmeans_step(zf, zbf, st, i, state_ref, total_ref, count_ref,
                     colsum_ref)

    @pl.when(t == _ITERS)
    def _final():
        zb = z_ref[...]
        z_sq = jnp.sum(zb * zb, axis=1, keepdims=True)
        sc = jax.lax.dot_general(
            zb, st, (((1,), (1,)), ((), ())),
            preferred_element_type=jnp.float32)
        n0 = jnp.sqrt(jnp.maximum(
            z_sq - 2.0 * sc[:, 0:1] + jnp.sum(st[0:1, :] * st[0:1, :]), 0.0))
        n1 = jnp.sqrt(jnp.maximum(
            z_sq - 2.0 * sc[:, 1:2] + jnp.sum(st[1:2, :] * st[1:2, :]), 0.0))
        n2 = jnp.sqrt(jnp.maximum(
            z_sq - 2.0 * sc[:, 2:3] + jnp.sum(st[2:3, :] * st[2:3, :]), 0.0))
        measure = jnp.minimum(jnp.minimum(n0, n1), n2)
        m2 = measure * measure
        c_base = 2.0 * _D
        beta = jnp.zeros_like(m2)
        for scale in (0.1, 0.2, 0.5, 1.0, 2.0, 5.0, 10.0):
            c = c_base * scale
            beta = beta + c / (c + m2)
        pen_ref[...] = _BAND * jnp.log(jnp.abs(beta) + _EPS)

        u = u_ref[...]
        w = w_ref[...]
        uw = jnp.sum(u * w)
        muw = -1.0 + jax.nn.softplus(uw)
        uhat = u + (muw - uw) * w / jnp.sum(w * w)
        zwb = jnp.sum(zb * w, axis=1, keepdims=True) + b_ref[0, 0]
        th = jnp.tanh(zwb)
        fz_ref[...] = zb + th * uhat
        wu = jnp.sum(w * uhat)
        psi_u = (1.0 - th * th) * wu
        ld_ref[...] = jnp.log(jnp.abs(1.0 + psi_u) + _EPS)


@functools.partial(jax.jit, static_argnames=("interpret",))
def _run(z, u, w, b, interpret=False):
    p = jax.random.uniform(jax.random.key(42), (z.shape[0],),
                           minval=0.0, maxval=1.0)
    _, ind = jax.lax.top_k(p, _N_CLUSTERS)
    state0 = jnp.take(z, ind, axis=0)

    u2 = u.reshape(1, _D)
    w2 = w.reshape(1, _D)
    b2 = b.reshape(1, 1)

    def _const_map(t, i):
        return (0, 0)

    def _z_map(t, i):
        return (jnp.where((t == 0) | (t == _ITERS), i, 0), 0)

    def _out_map(t, i):
        return (jnp.where(t == _ITERS, i, 0), 0)

    fz, ld, pen = pl.pallas_call(
        _body,
        grid=(_ITERS + 1, _NB),
        in_specs=[
            pl.BlockSpec((_N_CLUSTERS, _D), _const_map),
            pl.BlockSpec((1, _D), _const_map),
            pl.BlockSpec((1, _D), _const_map),
            pl.BlockSpec((1, 1), _const_map),
            pl.BlockSpec((_BR, _D), _z_map),
        ],
        out_specs=[
            pl.BlockSpec((_BR, _D), _out_map),
            pl.BlockSpec((_BR, 1), _out_map),
            pl.BlockSpec((_BR, 1), _out_map),
        ],
        out_shape=[
            jax.ShapeDtypeStruct((_BATCH, _D), jnp.float32),
            jax.ShapeDtypeStruct((_BATCH, 1), jnp.float32),
            jax.ShapeDtypeStruct((_BATCH, 1), jnp.float32),
        ],
        scratch_shapes=[
            pltpu.VMEM((_N_CLUSTERS, _D), jnp.float32),
            pltpu.VMEM((_N_CLUSTERS, _D), jnp.float32),
            pltpu.VMEM((_N_CLUSTERS, 128), jnp.float32),
            pltpu.VMEM((1, _D), jnp.float32),
            pltpu.VMEM((_NB, _BR, _D), jnp.bfloat16),
        ],
        compiler_params=pltpu.CompilerParams(
            dimension_semantics=("arbitrary", "arbitrary"),
        ),
        interpret=interpret,
    )(state0, u2, w2, b2, z)
    return fz, ld.reshape(-1), pen.reshape(-1)


def kernel(z, u, w, b):
    return _run(z, u, w, b)
